# baseline (speedup 1.0000x reference)
import jax
import jax.numpy as jnp
from jax.experimental import pallas as pl
from jax.experimental.pallas import tpu as pltpu

_NQ = 8


def _body(x_hbm, out_hbm, sem):
    B = x_hbm.shape[0]
    nb = B // _NQ
    for k in range(_NQ):
        pltpu.make_async_copy(
            x_hbm.at[pl.ds(k * nb, nb)], out_hbm.at[pl.ds(k * nb, nb)], sem.at[k]
        ).start()
    for k in range(_NQ):
        pltpu.make_async_copy(
            x_hbm.at[pl.ds(k * nb, nb)], out_hbm.at[pl.ds(k * nb, nb)], sem.at[k]
        ).wait()


def kernel(x, id, W):
    return pl.pallas_call(
        _body,
        in_specs=[pl.BlockSpec(memory_space=pltpu.MemorySpace.HBM)],
        out_specs=pl.BlockSpec(memory_space=pltpu.MemorySpace.HBM),
        out_shape=jax.ShapeDtypeStruct(x.shape, x.dtype),
        scratch_shapes=[pltpu.SemaphoreType.DMA((_NQ,))],
    )(x)


# speedup vs baseline: 43.5148x; 43.5148x over previous
import functools

import jax
import jax.numpy as jnp
from jax import lax
from jax.experimental import pallas as pl
from jax.experimental.pallas import tpu as pltpu
from jax.experimental.pallas import tpu_sc as plsc

_HB = 8


def _gather_sc(W, idx):
    info = plsc.get_sparse_core_info()
    nw = info.num_cores * info.num_subcores
    b = idx.shape[0]
    d = W.shape[1]
    b_per_w = b // nw
    smesh = plsc.VectorSubcoreMesh(core_axis_name="c", subcore_axis_name="s")

    @functools.partial(
        pl.kernel,
        mesh=smesh,
        out_type=jax.ShapeDtypeStruct((b, d), jnp.float32),
        scratch_types=[
            pltpu.VMEM((b_per_w,), jnp.int32),
            pltpu.VMEM((b_per_w, d), jnp.float32),
            pltpu.SemaphoreType.DMA,
        ],
    )
    def gat(table_hbm, idx_hbm, out_hbm, idx_v, rows_v, sem):
        wid = lax.axis_index("s") * info.num_cores + lax.axis_index("c")
        base = wid * b_per_w
        pltpu.sync_copy(idx_hbm.at[pl.ds(base, b_per_w)], idx_v)
        pltpu.async_copy(table_hbm.at[idx_v], rows_v, sem).wait()
        pltpu.sync_copy(rows_v, out_hbm.at[pl.ds(base, b_per_w)])

    return gat(W, idx)


def _assemble_body(x_ref, e_ref, out_ref):
    c = pl.program_id(2)

    @pl.when(c < 3)
    def _():
        out_ref[...] = x_ref[...]

    @pl.when(c == 3)
    def _():
        out_ref[0, 0] = e_ref[...]


def kernel(x, id, W):
    b, c, s, h, _ = x.shape
    hb = h // _HB
    rows = _gather_sc(W, id)
    emb3 = jnp.transpose(rows.reshape(b, h, h), (1, 2, 0))
    x_t = jnp.transpose(x, (1, 2, 3, 4, 0))
    out_t = pl.pallas_call(
        _assemble_body,
        grid=(_HB, s, c + 1),
        in_specs=[
            pl.BlockSpec(
                (1, 1, hb, h, b),
                lambda i, j, k: (jnp.minimum(k, 2), j, i, 0, 0),
            ),
            pl.BlockSpec((hb, h, b), lambda i, j, k: (i, 0, 0)),
        ],
        out_specs=pl.BlockSpec(
            (1, 1, hb, h, b), lambda i, j, k: (k, j, i, 0, 0)
        ),
        out_shape=jax.ShapeDtypeStruct((c + 1, s, h, h, b), x.dtype),
    )(x_t, emb3)
    return jnp.transpose(out_t, (4, 0, 1, 2, 3))


# speedup vs baseline: 73.0079x; 1.6778x over previous
import functools

import jax
import jax.numpy as jnp
from jax import lax
from jax.experimental import pallas as pl
from jax.experimental.pallas import tpu as pltpu
from jax.experimental.pallas import tpu_sc as plsc

_HB = 1


def _gather_sc(W, idx):
    info = plsc.get_sparse_core_info()
    nw = info.num_cores * info.num_subcores
    b = idx.shape[0]
    d = W.shape[1]
    b_per_w = b // nw
    smesh = plsc.VectorSubcoreMesh(core_axis_name="c", subcore_axis_name="s")

    @functools.partial(
        pl.kernel,
        mesh=smesh,
        out_type=jax.ShapeDtypeStruct((b, d), jnp.float32),
        scratch_types=[
            pltpu.VMEM((b_per_w,), jnp.int32),
            pltpu.VMEM((b_per_w, d), jnp.float32),
            pltpu.SemaphoreType.DMA,
        ],
    )
    def gat(table_hbm, idx_hbm, out_hbm, idx_v, rows_v, sem):
        wid = lax.axis_index("s") * info.num_cores + lax.axis_index("c")
        base = wid * b_per_w
        pltpu.sync_copy(idx_hbm.at[pl.ds(base, b_per_w)], idx_v)
        pltpu.async_copy(table_hbm.at[idx_v], rows_v, sem).wait()
        pltpu.sync_copy(rows_v, out_hbm.at[pl.ds(base, b_per_w)])

    return gat(W, idx)


def _assemble_body(x_ref, e_ref, out_ref):
    c = pl.program_id(2)

    @pl.when(c < 3)
    def _():
        out_ref[...] = x_ref[...]

    @pl.when(c == 3)
    def _():
        out_ref[0, 0] = e_ref[...]


def kernel(x, id, W):
    b, c, s, h, _ = x.shape
    hb = h // _HB
    rows = _gather_sc(W, id)
    emb3 = jnp.transpose(rows.reshape(b, h, h), (1, 2, 0))
    x_t = jnp.transpose(x, (1, 2, 3, 4, 0))
    out_t = pl.pallas_call(
        _assemble_body,
        grid=(_HB, s, c + 1),
        in_specs=[
            pl.BlockSpec(
                (1, 1, hb, h, b),
                lambda i, j, k: (jnp.minimum(k, 2), j, i, 0, 0),
            ),
            pl.BlockSpec((hb, h, b), lambda i, j, k: (i, 0, 0)),
        ],
        out_specs=pl.BlockSpec(
            (1, 1, hb, h, b), lambda i, j, k: (k, j, i, 0, 0)
        ),
        out_shape=jax.ShapeDtypeStruct((c + 1, s, h, h, b), x.dtype),
    )(x_t, emb3)
    return jnp.transpose(out_t, (4, 0, 1, 2, 3))


# speedup vs baseline: 75.7004x; 1.0369x over previous
import functools

import jax
import jax.numpy as jnp
from jax import lax
from jax.experimental import pallas as pl
from jax.experimental.pallas import tpu as pltpu
from jax.experimental.pallas import tpu_sc as plsc

_HB = 1


def _gather_sc(W, idx):
    info = plsc.get_sparse_core_info()
    nw = info.num_cores * info.num_subcores
    b = idx.shape[0]
    d = W.shape[1]
    b_per_w = b // nw
    smesh = plsc.VectorSubcoreMesh(core_axis_name="c", subcore_axis_name="s")

    @functools.partial(
        pl.kernel,
        mesh=smesh,
        out_type=jax.ShapeDtypeStruct((b, d), jnp.float32),
        scratch_types=[
            pltpu.VMEM((b_per_w,), jnp.int32),
            pltpu.VMEM((b_per_w, d), jnp.float32),
            pltpu.SemaphoreType.DMA,
        ],
    )
    def gat(table_hbm, idx_hbm, out_hbm, idx_v, rows_v, sem):
        wid = lax.axis_index("s") * info.num_cores + lax.axis_index("c")
        base = wid * b_per_w
        pltpu.sync_copy(idx_hbm.at[pl.ds(base, b_per_w)], idx_v)
        pltpu.async_copy(table_hbm.at[idx_v], rows_v, sem).wait()
        pltpu.sync_copy(rows_v, out_hbm.at[pl.ds(base, b_per_w)])

    return gat(W, idx)


def _assemble_body(x_ref, e_ref, out_ref):
    c = pl.program_id(1)

    @pl.when(c < 3)
    def _():
        out_ref[...] = x_ref[...]

    @pl.when(c == 3)
    def _():
        out_ref[0, 0] = e_ref[...]


def kernel(x, id, W):
    b, c, s, h, _ = x.shape
    hb = h // _HB
    rows = _gather_sc(W, id)
    emb3 = jnp.transpose(rows.reshape(b, h, h), (1, 2, 0))
    x_t = jnp.transpose(x, (1, 2, 3, 4, 0))
    out_t = pl.pallas_call(
        _assemble_body,
        grid=(_HB, c + 1, s),
        in_specs=[
            pl.BlockSpec(
                (1, 1, hb, h, b),
                lambda i, k, j: (
                    jnp.minimum(k, 2),
                    jnp.where(k < 3, j, s - 1),
                    i,
                    0,
                    0,
                ),
            ),
            pl.BlockSpec((hb, h, b), lambda i, k, j: (i, 0, 0)),
        ],
        out_specs=pl.BlockSpec(
            (1, 1, hb, h, b), lambda i, k, j: (k, j, i, 0, 0)
        ),
        out_shape=jax.ShapeDtypeStruct((c + 1, s, h, h, b), x.dtype),
    )(x_t, emb3)
    return jnp.transpose(out_t, (4, 0, 1, 2, 3))
